# baseline (speedup 1.0000x reference)
import functools

import jax
import jax.numpy as jnp
from jax import lax
from jax.experimental import pallas as pl
from jax.experimental.pallas import tpu as pltpu
from jax.experimental.pallas import tpu_sc as plsc

T, N, D = 4, 10000, 128
E = 320000

NUM_SC = 2
NUM_TILES = 16
T_PER_SC = T // NUM_SC
E_PER_TILE = E // NUM_TILES
CHUNK = 80
NCH = E_PER_TILE // CHUNK
NSLOT = 4
ZB_ROWS = 40
ROWS_MAJOR = 640
ROWS_LAST = N - (NUM_TILES - 1) * ROWS_MAJOR


def _mm_body(x_ref, w_ref, o_ref):
    o_ref[...] = jnp.dot(x_ref[...], w_ref[...],
                         preferred_element_type=jnp.float32)


def _project(inputs_flat, weight):
    bn = 2000
    grid = (inputs_flat.shape[0] // bn,)
    return pl.pallas_call(
        _mm_body,
        grid=grid,
        in_specs=[
            pl.BlockSpec((bn, inputs_flat.shape[1]), lambda i: (i, 0)),
            pl.BlockSpec(weight.shape, lambda i: (0, 0)),
        ],
        out_specs=pl.BlockSpec((bn, D), lambda i: (i, 0)),
        out_shape=jax.ShapeDtypeStruct((inputs_flat.shape[0], D), jnp.float32),
    )(inputs_flat, weight)


def _relu_body(x_ref, o_ref):
    o_ref[...] = jnp.maximum(x_ref[...], 0.0)


def _relu(x_flat):
    bn = 2000
    grid = (x_flat.shape[0] // bn,)
    return pl.pallas_call(
        _relu_body,
        grid=grid,
        in_specs=[pl.BlockSpec((bn, D), lambda i: (i, 0))],
        out_specs=pl.BlockSpec((bn, D), lambda i: (i, 0)),
        out_shape=jax.ShapeDtypeStruct(x_flat.shape, jnp.float32),
    )(x_flat)


def _sc_body(support_hbm, rows_hbm, cols_hbm, out_hbm,
             rows_v, cols_v, gath_v, zb_v, acc_sh,
             sem_idx, semg, sems, semz):
    c = lax.axis_index("c")
    s = lax.axis_index("s")

    zeros16 = jnp.zeros((16,), jnp.float32)

    def _zb_zero(r, _):
        for j in range(D // 16):
            zb_v[r, pl.ds(j * 16, 16)] = zeros16
        return 0
    lax.fori_loop(0, ZB_ROWS, _zb_zero, 0)

    wb_base = s * ROWS_MAJOR
    n_zero = jnp.where(s < NUM_TILES - 1,
                       ROWS_MAJOR // ZB_ROWS, ROWS_LAST // ZB_ROWS)

    for ti in range(T_PER_SC):
        t = c * T_PER_SC + ti
        ebase = t * E + s * E_PER_TILE

        def _idx_load(i, j):
            off = pl.multiple_of(ebase + i * CHUNK, 8)
            pltpu.async_copy(rows_hbm.at[pl.ds(off, CHUNK)], rows_v[j],
                             sem_idx[j])
            pltpu.async_copy(cols_hbm.at[pl.ds(off, CHUNK)], cols_v[j],
                             sem_idx[j])

        def _idx_wait(i, j):
            off = pl.multiple_of(ebase + i * CHUNK, 8)
            pltpu.make_async_copy(rows_hbm.at[pl.ds(off, CHUNK)], rows_v[j],
                                  sem_idx[j]).wait()
            pltpu.make_async_copy(cols_hbm.at[pl.ds(off, CHUNK)], cols_v[j],
                                  sem_idx[j]).wait()

        def _gather(j):
            pltpu.async_copy(support_hbm.at[cols_v[j]], gath_v[j], semg[j])

        def _gather_wait(j):
            pltpu.make_async_copy(support_hbm.at[cols_v[j]], gath_v[j],
                                  semg[j]).wait()

        def _scatter(j):
            pltpu.async_copy(gath_v[j], acc_sh.at[rows_v[j]], sems[j],
                             add=True)

        def _scatter_wait(j):
            pltpu.make_async_copy(gath_v[j], acc_sh.at[rows_v[j]],
                                  sems[j]).wait()

        def _zero(p, _):
            r0 = wb_base + p * ZB_ROWS
            pltpu.async_copy(zb_v, acc_sh.at[pl.ds(r0, ZB_ROWS)], semz)
            return 0
        lax.fori_loop(0, n_zero, _zero, 0)

        def _zero_drain(p, _):
            r0 = wb_base + p * ZB_ROWS
            pltpu.make_async_copy(zb_v, acc_sh.at[pl.ds(r0, ZB_ROWS)],
                                  semz).wait()
            return 0
        lax.fori_loop(0, n_zero, _zero_drain, 0)
        plsc.subcore_barrier()

        _idx_load(0, 0)
        _idx_load(1, 1)
        _idx_wait(0, 0)
        _gather(0)

        def _ring(k, _):
            for b in range(NSLOT):
                i = 4 * k + b
                ja = (b + 2) % NSLOT
                jb = (b + 1) % NSLOT
                _gather_wait(b)
                _scatter(b)
                if b >= 2:
                    _scatter_wait(ja)
                else:
                    @pl.when(k >= 1)
                    def _():
                        _scatter_wait(ja)
                _idx_load(i + 2, ja)
                _idx_wait(i + 1, jb)
                _gather(jb)
            return 0
        lax.fori_loop(0, (NCH - 2) // NSLOT, _ring, 0)

        _gather_wait(0)
        _scatter(0)
        _scatter_wait(2)
        _idx_wait(NCH - 1, 1)
        _gather(1)
        _gather_wait(1)
        _scatter(1)
        _scatter_wait(3)
        _scatter_wait(0)
        _scatter_wait(1)
        plsc.subcore_barrier()

        @pl.when(s < NUM_TILES - 1)
        def _():
            pltpu.sync_copy(acc_sh.at[pl.ds(wb_base, ROWS_MAJOR)],
                            out_hbm.at[t, pl.ds(wb_base, ROWS_MAJOR), :])

        @pl.when(s == NUM_TILES - 1)
        def _():
            pltpu.sync_copy(acc_sh.at[pl.ds(wb_base, ROWS_LAST)],
                            out_hbm.at[t, pl.ds(wb_base, ROWS_LAST), :])


def _aggregate(support_flat, rows, cols):
    mesh = plsc.VectorSubcoreMesh(core_axis_name="c", subcore_axis_name="s")
    f = functools.partial(
        pl.kernel,
        out_type=jax.ShapeDtypeStruct((T, N, D), jnp.float32),
        mesh=mesh,
        scratch_types=[
            [pltpu.VMEM((CHUNK,), jnp.int32) for _ in range(NSLOT)],
            [pltpu.VMEM((CHUNK,), jnp.int32) for _ in range(NSLOT)],
            [pltpu.VMEM((CHUNK, D), jnp.float32) for _ in range(NSLOT)],
            pltpu.VMEM((ZB_ROWS, D), jnp.float32),
            pltpu.VMEM_SHARED((N, D), jnp.float32),
            [pltpu.SemaphoreType.DMA for _ in range(NSLOT)],
            [pltpu.SemaphoreType.DMA for _ in range(NSLOT)],
            [pltpu.SemaphoreType.DMA for _ in range(NSLOT)],
            pltpu.SemaphoreType.DMA,
        ],
    )(_sc_body)
    return f(support_flat, rows, cols)


def kernel(inputs, edge_index, weight):
    inputs_flat = inputs.reshape(T * N, inputs.shape[-1])
    support_flat = _project(inputs_flat, weight)
    rows = edge_index[:, 0, :].astype(jnp.int32).reshape(T * E)
    toff = (jnp.arange(T, dtype=jnp.int64) * N)[:, None]
    cols = (edge_index[:, 1, :] + toff).astype(jnp.int32).reshape(T * E)
    acc = _aggregate(support_flat, rows, cols)
    return _relu(acc.reshape(T * N, D)).reshape(T, N, D)


# speedup vs baseline: 1.4251x; 1.4251x over previous
import functools

import jax
import jax.numpy as jnp
from jax import lax
from jax.experimental import pallas as pl
from jax.experimental.pallas import tpu as pltpu
from jax.experimental.pallas import tpu_sc as plsc

T, N, D = 4, 10000, 128
E = 320000

NUM_SC = 2
NUM_TILES = 16
T_PER_SC = T // NUM_SC
E_PER_TILE = E // NUM_TILES
CHUNK = 80
NSLOT = 4
BLK_EDGES = 4000
N_BLKS = E_PER_TILE // BLK_EDGES
BLK_CHUNKS = BLK_EDGES // CHUNK
ZB_ROWS = CHUNK
ROWS_MAJOR = 640
ROWS_LAST = N - (NUM_TILES - 1) * ROWS_MAJOR


def _mm_body(x_ref, w_ref, o_ref):
    o_ref[...] = jnp.dot(x_ref[...], w_ref[...],
                         preferred_element_type=jnp.float32)


def _project(inputs_flat, weight):
    bn = 2000
    grid = (inputs_flat.shape[0] // bn,)
    return pl.pallas_call(
        _mm_body,
        grid=grid,
        in_specs=[
            pl.BlockSpec((bn, inputs_flat.shape[1]), lambda i: (i, 0)),
            pl.BlockSpec(weight.shape, lambda i: (0, 0)),
        ],
        out_specs=pl.BlockSpec((bn, D), lambda i: (i, 0)),
        out_shape=jax.ShapeDtypeStruct((inputs_flat.shape[0], D), jnp.float32),
    )(inputs_flat, weight)


def _relu_body(x_ref, o_ref):
    o_ref[...] = jnp.maximum(x_ref[...], 0.0)


def _relu(x_flat):
    bn = 2000
    grid = (x_flat.shape[0] // bn,)
    return pl.pallas_call(
        _relu_body,
        grid=grid,
        in_specs=[pl.BlockSpec((bn, D), lambda i: (i, 0))],
        out_specs=pl.BlockSpec((bn, D), lambda i: (i, 0)),
        out_shape=jax.ShapeDtypeStruct(x_flat.shape, jnp.float32),
    )(x_flat)


def _sc_body(support_hbm, rows_hbm, cols_hbm, out_hbm,
             rows_st, cols_st, rows_v, gath_v, acc_sh,
             semg, sems, semz):
    c = lax.axis_index("c")
    s = lax.axis_index("s")

    zeros16 = jnp.zeros((16,), jnp.float32)
    wb_base = s * ROWS_MAJOR
    n_zero = jnp.where(s < NUM_TILES - 1,
                       ROWS_MAJOR // ZB_ROWS, ROWS_LAST // ZB_ROWS)

    def _col_slice(i):
        off = pl.multiple_of(i * CHUNK, 8)
        return cols_st.at[pl.ds(off, CHUNK)]

    def _gather(i, j):
        pltpu.async_copy(support_hbm.at[_col_slice(i)], gath_v[j], semg[j])

    def _gather_wait(i, j):
        pltpu.make_async_copy(support_hbm.at[_col_slice(i)], gath_v[j],
                              semg[j]).wait()

    def _scatter(j):
        pltpu.async_copy(gath_v[j], acc_sh.at[rows_v[j]], sems[j], add=True)

    def _scatter_wait(j):
        pltpu.make_async_copy(gath_v[j], acc_sh.at[rows_v[j]],
                              sems[j]).wait()

    def _prep_rows(i, j):
        for q in range(CHUNK // 16):
            off = pl.multiple_of(i * CHUNK + q * 16, 8)
            rows_v[j][pl.ds(q * 16, 16)] = rows_st[pl.ds(off, 16)]

    for ti in range(T_PER_SC):
        t = c * T_PER_SC + ti
        ebase = t * E + s * E_PER_TILE

        def _gz(r, _):
            for q in range(D // 16):
                gath_v[0][r, pl.ds(q * 16, 16)] = zeros16
            return 0
        lax.fori_loop(0, ZB_ROWS, _gz, 0)

        def _zero(p, _):
            r0 = wb_base + p * ZB_ROWS
            pltpu.async_copy(gath_v[0], acc_sh.at[pl.ds(r0, ZB_ROWS)], semz)
            return 0
        lax.fori_loop(0, n_zero, _zero, 0)

        def _zero_drain(p, _):
            r0 = wb_base + p * ZB_ROWS
            pltpu.make_async_copy(gath_v[0], acc_sh.at[pl.ds(r0, ZB_ROWS)],
                                  semz).wait()
            return 0
        lax.fori_loop(0, n_zero, _zero_drain, 0)
        plsc.subcore_barrier()

        for blk in range(N_BLKS):
            bbase = ebase + blk * BLK_EDGES
            pltpu.sync_copy(rows_hbm.at[pl.ds(bbase, BLK_EDGES)], rows_st)
            pltpu.sync_copy(cols_hbm.at[pl.ds(bbase, BLK_EDGES)], cols_st)

            _gather(0, 0)
            _gather(1, 1)

            def _ring(k, _):
                for b in range(NSLOT):
                    i = 4 * k + b
                    ja = (b + 2) % NSLOT
                    _prep_rows(i, b)
                    if b >= 2:
                        _scatter_wait(ja)
                    else:
                        @pl.when(k >= 1)
                        def _():
                            _scatter_wait(ja)
                    _gather(i + 2, ja)
                    _gather_wait(i, b)
                    _scatter(b)
                return 0
            lax.fori_loop(0, (BLK_CHUNKS - 2) // NSLOT, _ring, 0)

            _prep_rows(BLK_CHUNKS - 2, 0)
            _scatter_wait(2)
            _gather_wait(BLK_CHUNKS - 2, 0)
            _scatter(0)
            _prep_rows(BLK_CHUNKS - 1, 1)
            _scatter_wait(3)
            _gather_wait(BLK_CHUNKS - 1, 1)
            _scatter(1)
            _scatter_wait(0)
            _scatter_wait(1)
        plsc.subcore_barrier()

        @pl.when(s < NUM_TILES - 1)
        def _():
            pltpu.sync_copy(acc_sh.at[pl.ds(wb_base, ROWS_MAJOR)],
                            out_hbm.at[t, pl.ds(wb_base, ROWS_MAJOR), :])

        @pl.when(s == NUM_TILES - 1)
        def _():
            pltpu.sync_copy(acc_sh.at[pl.ds(wb_base, ROWS_LAST)],
                            out_hbm.at[t, pl.ds(wb_base, ROWS_LAST), :])


def _aggregate(support_flat, rows, cols):
    mesh = plsc.VectorSubcoreMesh(core_axis_name="c", subcore_axis_name="s")
    f = functools.partial(
        pl.kernel,
        out_type=jax.ShapeDtypeStruct((T, N, D), jnp.float32),
        mesh=mesh,
        scratch_types=[
            pltpu.VMEM((BLK_EDGES,), jnp.int32),
            pltpu.VMEM((BLK_EDGES,), jnp.int32),
            [pltpu.VMEM((CHUNK,), jnp.int32) for _ in range(NSLOT)],
            [pltpu.VMEM((CHUNK, D), jnp.float32) for _ in range(NSLOT)],
            pltpu.VMEM_SHARED((N, D), jnp.float32),
            [pltpu.SemaphoreType.DMA for _ in range(NSLOT)],
            [pltpu.SemaphoreType.DMA for _ in range(NSLOT)],
            pltpu.SemaphoreType.DMA,
        ],
    )(_sc_body)
    return f(support_flat, rows, cols)


def kernel(inputs, edge_index, weight):
    inputs_flat = inputs.reshape(T * N, inputs.shape[-1])
    support_flat = _project(inputs_flat, weight)
    rows = edge_index[:, 0, :].astype(jnp.int32).reshape(T * E)
    toff = (jnp.arange(T, dtype=jnp.int64) * N)[:, None]
    cols = (edge_index[:, 1, :] + toff).astype(jnp.int32).reshape(T * E)
    acc = _aggregate(support_flat, rows, cols)
    return _relu(acc.reshape(T * N, D)).reshape(T, N, D)


# speedup vs baseline: 1.5062x; 1.0569x over previous
import functools

import jax
import jax.numpy as jnp
from jax import lax
from jax.experimental import pallas as pl
from jax.experimental.pallas import tpu as pltpu
from jax.experimental.pallas import tpu_sc as plsc

T, N, D = 4, 10000, 128
E = 320000

NUM_SC = 2
NUM_TILES = 16
T_PER_SC = T // NUM_SC
E_PER_TILE = E // NUM_TILES
CHUNK = 80
NSLOT = 4
BLK_EDGES = 4000
N_BLKS = E_PER_TILE // BLK_EDGES
BLK_CHUNKS = BLK_EDGES // CHUNK
ZB_ROWS = CHUNK
ROWS_MAJOR = 640
ROWS_LAST = N - (NUM_TILES - 1) * ROWS_MAJOR


def _mm_body(x_ref, w_ref, o_ref):
    o_ref[...] = jnp.dot(x_ref[...], w_ref[...],
                         preferred_element_type=jnp.float32)


def _project(inputs_flat, weight):
    bn = 2000
    grid = (inputs_flat.shape[0] // bn,)
    return pl.pallas_call(
        _mm_body,
        grid=grid,
        in_specs=[
            pl.BlockSpec((bn, inputs_flat.shape[1]), lambda i: (i, 0)),
            pl.BlockSpec(weight.shape, lambda i: (0, 0)),
        ],
        out_specs=pl.BlockSpec((bn, D), lambda i: (i, 0)),
        out_shape=jax.ShapeDtypeStruct((inputs_flat.shape[0], D), jnp.float32),
    )(inputs_flat, weight)


def _relu_body(x_ref, o_ref):
    o_ref[...] = jnp.maximum(x_ref[...], 0.0)


def _relu(x_flat):
    bn = 2000
    grid = (x_flat.shape[0] // bn,)
    return pl.pallas_call(
        _relu_body,
        grid=grid,
        in_specs=[pl.BlockSpec((bn, D), lambda i: (i, 0))],
        out_specs=pl.BlockSpec((bn, D), lambda i: (i, 0)),
        out_shape=jax.ShapeDtypeStruct(x_flat.shape, jnp.float32),
    )(x_flat)


def _sc_body(support_hbm, rows_hbm, cols_hbm, out_hbm,
             rows_st, cols_st, rows_v, gath_v, acc_sh,
             semg, sems, semz):
    c = lax.axis_index("c")
    s = lax.axis_index("s")

    zeros16 = jnp.zeros((16,), jnp.float32)
    wb_base = s * ROWS_MAJOR
    n_zero = jnp.where(s < NUM_TILES - 1,
                       ROWS_MAJOR // ZB_ROWS, ROWS_LAST // ZB_ROWS)

    def _col_slice(i):
        off = pl.multiple_of(i * CHUNK, 8)
        return cols_st.at[pl.ds(off, CHUNK)]

    def _gather(i, j):
        pltpu.async_copy(support_hbm.at[_col_slice(i)], gath_v[j], semg[j])

    def _gather_wait(i, j):
        pltpu.make_async_copy(support_hbm.at[_col_slice(i)], gath_v[j],
                              semg[j]).wait()

    def _scatter(j):
        pltpu.async_copy(gath_v[j], acc_sh.at[rows_v[j]], sems[j], add=True)

    def _scatter_wait(j):
        pltpu.make_async_copy(gath_v[j], acc_sh.at[rows_v[j]],
                              sems[j]).wait()

    def _prep_rows(i, j):
        for q in range(CHUNK // 16):
            off = pl.multiple_of(i * CHUNK + q * 16, 8)
            rows_v[j][pl.ds(q * 16, 16)] = rows_st[pl.ds(off, 16)]

    for ti in range(T_PER_SC):
        t = c * T_PER_SC + ti
        ebase = t * E + s * E_PER_TILE

        def _gz(r, _):
            for q in range(D // 16):
                gath_v[0][r, pl.ds(q * 16, 16)] = zeros16
            return 0
        lax.fori_loop(0, ZB_ROWS, _gz, 0)

        def _zero(p, _):
            r0 = wb_base + p * ZB_ROWS
            pltpu.async_copy(gath_v[0], acc_sh.at[pl.ds(r0, ZB_ROWS)], semz)
            return 0
        lax.fori_loop(0, n_zero, _zero, 0)

        def _zero_drain(p, _):
            r0 = wb_base + p * ZB_ROWS
            pltpu.make_async_copy(gath_v[0], acc_sh.at[pl.ds(r0, ZB_ROWS)],
                                  semz).wait()
            return 0
        lax.fori_loop(0, n_zero, _zero_drain, 0)
        plsc.subcore_barrier()

        for blk in range(N_BLKS):
            bbase = ebase + blk * BLK_EDGES
            pltpu.sync_copy(rows_hbm.at[pl.ds(bbase, BLK_EDGES)], rows_st)
            pltpu.sync_copy(cols_hbm.at[pl.ds(bbase, BLK_EDGES)], cols_st)

            _gather(0, 0)
            _gather(1, 1)
            _gather(2, 2)

            def _ring(k, _):
                for b in range(NSLOT):
                    i = 4 * k + b
                    ja = (b + 3) % NSLOT
                    _prep_rows(i, b)
                    if b >= 1:
                        _scatter_wait(ja)
                    else:
                        @pl.when(k >= 1)
                        def _():
                            _scatter_wait(ja)
                    _gather(i + 3, ja)
                    _gather_wait(i, b)
                    _scatter(b)
                return 0
            lax.fori_loop(0, (BLK_CHUNKS - 6) // NSLOT, _ring, 0)

            for i in range(BLK_CHUNKS - 6, BLK_CHUNKS):
                b = i % NSLOT
                _prep_rows(i, b)
                ja = (b + 3) % NSLOT
                _scatter_wait(ja)
                if i + 3 < BLK_CHUNKS:
                    _gather(i + 3, ja)
                _gather_wait(i, b)
                _scatter(b)
            _scatter_wait((BLK_CHUNKS - 1) % NSLOT)
        plsc.subcore_barrier()

        @pl.when(s < NUM_TILES - 1)
        def _():
            pltpu.sync_copy(acc_sh.at[pl.ds(wb_base, ROWS_MAJOR)],
                            out_hbm.at[t, pl.ds(wb_base, ROWS_MAJOR), :])

        @pl.when(s == NUM_TILES - 1)
        def _():
            pltpu.sync_copy(acc_sh.at[pl.ds(wb_base, ROWS_LAST)],
                            out_hbm.at[t, pl.ds(wb_base, ROWS_LAST), :])


def _aggregate(support_flat, rows, cols):
    mesh = plsc.VectorSubcoreMesh(core_axis_name="c", subcore_axis_name="s")
    f = functools.partial(
        pl.kernel,
        out_type=jax.ShapeDtypeStruct((T, N, D), jnp.float32),
        mesh=mesh,
        scratch_types=[
            pltpu.VMEM((BLK_EDGES,), jnp.int32),
            pltpu.VMEM((BLK_EDGES,), jnp.int32),
            [pltpu.VMEM((CHUNK,), jnp.int32) for _ in range(NSLOT)],
            [pltpu.VMEM((CHUNK, D), jnp.float32) for _ in range(NSLOT)],
            pltpu.VMEM_SHARED((N, D), jnp.float32),
            [pltpu.SemaphoreType.DMA for _ in range(NSLOT)],
            [pltpu.SemaphoreType.DMA for _ in range(NSLOT)],
            pltpu.SemaphoreType.DMA,
        ],
    )(_sc_body)
    return f(support_flat, rows, cols)


def kernel(inputs, edge_index, weight):
    inputs_flat = inputs.reshape(T * N, inputs.shape[-1])
    support_flat = _project(inputs_flat, weight)
    rows = edge_index[:, 0, :].astype(jnp.int32).reshape(T * E)
    toff = (jnp.arange(T, dtype=jnp.int64) * N)[:, None]
    cols = (edge_index[:, 1, :] + toff).astype(jnp.int32).reshape(T * E)
    acc = _aggregate(support_flat, rows, cols)
    return _relu(acc.reshape(T * N, D)).reshape(T, N, D)


# speedup vs baseline: 1.6087x; 1.0681x over previous
import functools

import jax
import jax.numpy as jnp
from jax import lax
from jax.experimental import pallas as pl
from jax.experimental.pallas import tpu as pltpu
from jax.experimental.pallas import tpu_sc as plsc

T, N, D = 4, 10000, 128
E = 320000

NUM_SC = 2
NUM_TILES = 16
T_PER_SC = T // NUM_SC
E_PER_TILE = E // NUM_TILES
CHUNK = 80
NSLOT = 4
GLA = 3
BLK_EDGES = 2000
N_BLKS = E_PER_TILE // BLK_EDGES
BLK_CHUNKS = BLK_EDGES // CHUNK
STEADY = (BLK_CHUNKS - 5) // NSLOT
ZB_ROWS = CHUNK
ROWS_MAJOR = 640
ROWS_LAST = N - (NUM_TILES - 1) * ROWS_MAJOR


def _mm_body(x_ref, w_ref, o_ref):
    o_ref[...] = jnp.dot(x_ref[...], w_ref[...],
                         preferred_element_type=jnp.float32)


def _project(inputs_flat, weight):
    bn = 2000
    grid = (inputs_flat.shape[0] // bn,)
    return pl.pallas_call(
        _mm_body,
        grid=grid,
        in_specs=[
            pl.BlockSpec((bn, inputs_flat.shape[1]), lambda i: (i, 0)),
            pl.BlockSpec(weight.shape, lambda i: (0, 0)),
        ],
        out_specs=pl.BlockSpec((bn, D), lambda i: (i, 0)),
        out_shape=jax.ShapeDtypeStruct((inputs_flat.shape[0], D), jnp.float32),
    )(inputs_flat, weight)


def _relu_body(x_ref, o_ref):
    o_ref[...] = jnp.maximum(x_ref[...], 0.0)


def _relu(x_flat):
    bn = 2000
    grid = (x_flat.shape[0] // bn,)
    return pl.pallas_call(
        _relu_body,
        grid=grid,
        in_specs=[pl.BlockSpec((bn, D), lambda i: (i, 0))],
        out_specs=pl.BlockSpec((bn, D), lambda i: (i, 0)),
        out_shape=jax.ShapeDtypeStruct(x_flat.shape, jnp.float32),
    )(x_flat)


def _sc_body(support_hbm, rows_hbm, cols_hbm, out_hbm,
             rows_st, cols_st, rows_v, gath_v, acc_sh,
             semg, sems, semz, semi):
    c = lax.axis_index("c")
    s = lax.axis_index("s")

    zeros16 = jnp.zeros((16,), jnp.float32)
    wb_base = s * ROWS_MAJOR
    n_zero = jnp.where(s < NUM_TILES - 1,
                       ROWS_MAJOR // ZB_ROWS, ROWS_LAST // ZB_ROWS)

    def _col_slice(buf, li):
        off = pl.multiple_of(li * CHUNK, 8)
        return cols_st[buf].at[pl.ds(off, CHUNK)]

    def _gather(buf, li, j):
        pltpu.async_copy(support_hbm.at[_col_slice(buf, li)], gath_v[j],
                         semg[j])

    def _gather_wait(buf, li, j):
        pltpu.make_async_copy(support_hbm.at[_col_slice(buf, li)], gath_v[j],
                              semg[j]).wait()

    def _scatter(j):
        pltpu.async_copy(gath_v[j], acc_sh.at[rows_v[j]], sems[j], add=True)

    def _scatter_wait(j):
        pltpu.make_async_copy(gath_v[j], acc_sh.at[rows_v[j]],
                              sems[j]).wait()

    def _prep_rows(buf, li, j):
        for q in range(CHUNK // 16):
            off = pl.multiple_of(li * CHUNK + q * 16, 8)
            rows_v[j][pl.ds(q * 16, 16)] = rows_st[buf][pl.ds(off, 16)]

    def _prefetch(ebase, blk):
        buf = blk % 2
        bbase = ebase + blk * BLK_EDGES
        pltpu.async_copy(rows_hbm.at[pl.ds(bbase, BLK_EDGES)], rows_st[buf],
                         semi[buf])
        pltpu.async_copy(cols_hbm.at[pl.ds(bbase, BLK_EDGES)], cols_st[buf],
                         semi[buf])

    def _prefetch_wait(ebase, blk):
        buf = blk % 2
        bbase = ebase + blk * BLK_EDGES
        pltpu.make_async_copy(rows_hbm.at[pl.ds(bbase, BLK_EDGES)],
                              rows_st[buf], semi[buf]).wait()
        pltpu.make_async_copy(cols_hbm.at[pl.ds(bbase, BLK_EDGES)],
                              cols_st[buf], semi[buf]).wait()

    for ti in range(T_PER_SC):
        t = c * T_PER_SC + ti
        ebase = t * E + s * E_PER_TILE

        _prefetch(ebase, 0)

        def _gz(r, _):
            for q in range(D // 16):
                gath_v[0][r, pl.ds(q * 16, 16)] = zeros16
            return 0
        lax.fori_loop(0, ZB_ROWS, _gz, 0)

        def _zero(p, _):
            r0 = wb_base + p * ZB_ROWS
            pltpu.async_copy(gath_v[0], acc_sh.at[pl.ds(r0, ZB_ROWS)], semz)
            return 0
        lax.fori_loop(0, n_zero, _zero, 0)

        def _zero_drain(p, _):
            r0 = wb_base + p * ZB_ROWS
            pltpu.make_async_copy(gath_v[0], acc_sh.at[pl.ds(r0, ZB_ROWS)],
                                  semz).wait()
            return 0
        lax.fori_loop(0, n_zero, _zero_drain, 0)
        _prefetch_wait(ebase, 0)

        for g in range(GLA):
            _gather(0, g, g)
        plsc.subcore_barrier()

        for blk in range(N_BLKS):
            base = blk * BLK_CHUNKS
            cur = blk % 2
            nxt = 1 - cur
            if blk + 1 < N_BLKS:
                _prefetch(ebase, blk + 1)

            def _ring(k, _):
                for b4 in range(NSLOT):
                    li = NSLOT * k + b4
                    jb = (base + b4) % NSLOT
                    ja = (jb + 3) % NSLOT
                    _prep_rows(cur, li, jb)
                    if blk == 0 and b4 == 0:
                        @pl.when(k >= 1)
                        def _():
                            _scatter_wait(ja)
                    else:
                        _scatter_wait(ja)
                    _gather(cur, li + GLA, ja)
                    _gather_wait(cur, li, jb)
                    _scatter(jb)
                return 0
            lax.fori_loop(0, STEADY, _ring, 0)

            for li in range(NSLOT * STEADY, BLK_CHUNKS):
                jb = (base + li) % NSLOT
                ja = (jb + 3) % NSLOT
                _prep_rows(cur, li, jb)
                _scatter_wait(ja)
                gl = li + GLA
                if gl < BLK_CHUNKS:
                    _gather(cur, gl, ja)
                elif blk + 1 < N_BLKS:
                    if gl == BLK_CHUNKS:
                        _prefetch_wait(ebase, blk + 1)
                    _gather(nxt, gl - BLK_CHUNKS, ja)
                _gather_wait(cur, li, jb)
                _scatter(jb)
        _scatter_wait((N_BLKS * BLK_CHUNKS - 1) % NSLOT)
        plsc.subcore_barrier()

        @pl.when(s < NUM_TILES - 1)
        def _():
            pltpu.sync_copy(acc_sh.at[pl.ds(wb_base, ROWS_MAJOR)],
                            out_hbm.at[t, pl.ds(wb_base, ROWS_MAJOR), :])

        @pl.when(s == NUM_TILES - 1)
        def _():
            pltpu.sync_copy(acc_sh.at[pl.ds(wb_base, ROWS_LAST)],
                            out_hbm.at[t, pl.ds(wb_base, ROWS_LAST), :])


def _aggregate(support_flat, rows, cols):
    mesh = plsc.VectorSubcoreMesh(core_axis_name="c", subcore_axis_name="s")
    f = functools.partial(
        pl.kernel,
        out_type=jax.ShapeDtypeStruct((T, N, D), jnp.float32),
        mesh=mesh,
        scratch_types=[
            [pltpu.VMEM((BLK_EDGES,), jnp.int32) for _ in range(2)],
            [pltpu.VMEM((BLK_EDGES,), jnp.int32) for _ in range(2)],
            [pltpu.VMEM((CHUNK,), jnp.int32) for _ in range(NSLOT)],
            [pltpu.VMEM((CHUNK, D), jnp.float32) for _ in range(NSLOT)],
            pltpu.VMEM_SHARED((N, D), jnp.float32),
            [pltpu.SemaphoreType.DMA for _ in range(NSLOT)],
            [pltpu.SemaphoreType.DMA for _ in range(NSLOT)],
            pltpu.SemaphoreType.DMA,
            [pltpu.SemaphoreType.DMA for _ in range(2)],
        ],
    )(_sc_body)
    return f(support_flat, rows, cols)


def kernel(inputs, edge_index, weight):
    inputs_flat = inputs.reshape(T * N, inputs.shape[-1])
    support_flat = _project(inputs_flat, weight)
    rows = edge_index[:, 0, :].astype(jnp.int32).reshape(T * E)
    toff = (jnp.arange(T, dtype=jnp.int64) * N)[:, None]
    cols = (edge_index[:, 1, :] + toff).astype(jnp.int32).reshape(T * E)
    acc = _aggregate(support_flat, rows, cols)
    return _relu(acc.reshape(T * N, D)).reshape(T, N, D)
